# baseline (speedup 1.0000x reference)
import jax
import jax.numpy as jnp
from jax.experimental import pallas as pl

_S = 8192
_B = 1024
_f32 = jnp.float32


def _sum32(a):
    acc = (a[0:8] + a[8:16]) + a[16:24]
    acc = acc + a[24:32]
    u = acc[0:4] + acc[4:8]
    v = u[0:2] + u[2:4]
    return v[0:1] + v[1:2]


def _km_block(x_ref, m_ref, s1_ref, s2_ref, c1r_ref, c1c_ref, c2r_ref,
              c2c_ref, out_ref):
    B = out_ref.shape[1]
    a1 = jnp.abs(s1_ref[...])
    a2 = jnp.abs(s2_ref[...])
    slo = jnp.minimum(a1, a2) + _f32(1e-6)
    shi = jnp.maximum(a1, a2) + _f32(1e-6)
    m = m_ref[...]

    es = []
    eb = []
    for k in range(8):
        xkb = jnp.broadcast_to(x_ref[k:k + 1, :], (32, B))
        d = xkb - jnp.broadcast_to(m[:, k:k + 1], (32, B))
        qs = d / jnp.broadcast_to(slo[:, k:k + 1], (32, B))
        qb = d / jnp.broadcast_to(shi[:, k:k + 1], (32, B))
        es.append(jnp.exp((qs * qs) * _f32(-0.5)))
        eb.append(jnp.exp((qb * qb) * _f32(-0.5)))

    def prod8(e):
        u0 = e[0] * e[4]
        u1 = e[1] * e[5]
        u2 = e[2] * e[6]
        u3 = e[3] * e[7]
        return (u0 * u2) * (u1 * u3)

    LL = prod8(es)
    UU = prod8(eb)

    subi = jax.lax.broadcasted_iota(jnp.int32, (32, 32), 0)
    lanei = jax.lax.broadcasted_iota(jnp.int32, (32, 32), 1)

    def onehot_perm(crow_ref, ccol_ref):
        crow = jnp.broadcast_to(crow_ref[...], (32, 32))
        ccol = jnp.broadcast_to(ccol_ref[...], (32, 32))
        lt = ccol < crow
        eq = ccol == crow
        cnt = jnp.where(lt | (eq & (subi < lanei)), 1, 0)
        rank_row = jnp.sum(cnt, axis=0, keepdims=True)
        P = jnp.broadcast_to(rank_row, (32, 32)) == subi
        return P

    P1 = onehot_perm(c1r_ref, c1c_ref)
    P2 = onehot_perm(c2r_ref, c2c_ref)

    def sorted_vals(P, crow_ref):
        cb = jnp.broadcast_to(crow_ref[...], (32, 32))
        return jnp.sum(jnp.where(P, cb, _f32(0)), axis=1, keepdims=True)

    c1s = sorted_vals(P1, c1r_ref)
    c2s = sorted_vals(P2, c2r_ref)

    def permute_rows(P, a):
        pf = P.astype(_f32)
        out = jnp.zeros((32, B), _f32)
        for l in range(32):
            out = out + (jnp.broadcast_to(pf[:, l:l + 1], (32, B)) *
                         jnp.broadcast_to(a[l:l + 1, :], (32, B)))
        return out

    L_UU = permute_rows(P1, UU)
    L_LL = permute_rows(P1, LL)
    R_UU = permute_rows(P2, UU)
    R_LL = permute_rows(P2, LL)

    c1s_b = jnp.broadcast_to(c1s, (32, B))
    c2s_b = jnp.broadcast_to(c2s, (32, B))
    s0 = _sum32(c1s_b * L_LL)
    t0 = _sum32(L_LL)
    s0r = _sum32(c2s_b * R_UU)
    t0r = _sum32(R_UU)
    dL = L_UU - L_LL
    dR = R_LL - R_UU
    ndL = c1s_b * dL
    ndR = c2s_b * dR
    q = s0 / t0
    qr = s0r / t0r

    cn = jnp.zeros((1, B), _f32)
    cd = jnp.zeros((1, B), _f32)
    cmn = jnp.full((1, B), jnp.inf, _f32)
    bestv = jnp.full((1, B), jnp.inf, _f32)
    besti = jnp.zeros((1, B), jnp.int32)
    for t in range(32):
        cn = cn + ndL[t:t + 1]
        cd = cd + dL[t:t + 1]
        ratio = (s0 + cn) / (t0 + cd)
        cmn = jnp.minimum(cmn, ratio)
        lout = jnp.minimum(cmn, q)
        keepv = (bestv < lout) | (bestv != bestv)
        keepi = keepv | (bestv == lout)
        bestv = jnp.where(keepv, bestv, lout)
        besti = jnp.where(keepi, besti, jnp.full((1, B), t, jnp.int32))
    L_loc = besti

    cn = jnp.zeros((1, B), _f32)
    cd = jnp.zeros((1, B), _f32)
    cmx = jnp.full((1, B), -jnp.inf, _f32)
    bestv = jnp.full((1, B), -jnp.inf, _f32)
    besti = jnp.zeros((1, B), jnp.int32)
    for t in range(32):
        cn = cn + ndR[t:t + 1]
        cd = cd + dR[t:t + 1]
        ratio = (s0r + cn) / (t0r + cd)
        cmx = jnp.maximum(cmx, ratio)
        rout = jnp.maximum(cmx, qr)
        keepv = (bestv > rout) | (bestv != bestv)
        keepi = keepv | (bestv == rout)
        bestv = jnp.where(keepv, bestv, rout)
        besti = jnp.where(keepi, besti, jnp.full((1, B), t, jnp.int32))
    R_loc = besti

    rowi = jax.lax.broadcasted_iota(jnp.int32, (32, B), 0)
    selL = jnp.where(rowi <= jnp.broadcast_to(L_loc, (32, B)), L_UU, L_LL)
    selR = jnp.where(rowi <= jnp.broadcast_to(R_loc, (32, B)), R_LL, R_UU)
    c1n_b = jnp.broadcast_to(c1c_ref[...], (32, B))
    c2n_b = jnp.broadcast_to(c2c_ref[...], (32, B))
    out_left = _sum32(c1n_b * selL) / _sum32(selL)
    out_right = _sum32(c2n_b * selR) / _sum32(selR)
    out_ref[...] = (out_right + out_left) / _f32(2.0)


def kernel(input_data, FRB_weights, c1, c2):
    xT = input_data.T
    m = FRB_weights[0:256].reshape(32, 8)
    s1 = FRB_weights[1:257].reshape(32, 8)
    s2 = FRB_weights[2:258].reshape(32, 8)
    c1r = c1.reshape(1, 32)
    c1c = c1.reshape(32, 1)
    c2r = c2.reshape(1, 32)
    c2c = c2.reshape(32, 1)
    rep = pl.BlockSpec((32, 8), lambda i: (0, 0))
    out = pl.pallas_call(
        _km_block,
        grid=(_S // _B,),
        in_specs=[
            pl.BlockSpec((8, _B), lambda i: (0, i)),
            rep, rep, rep,
            pl.BlockSpec((1, 32), lambda i: (0, 0)),
            pl.BlockSpec((32, 1), lambda i: (0, 0)),
            pl.BlockSpec((1, 32), lambda i: (0, 0)),
            pl.BlockSpec((32, 1), lambda i: (0, 0)),
        ],
        out_specs=pl.BlockSpec((1, _B), lambda i: (0, i)),
        out_shape=jax.ShapeDtypeStruct((1, _S), jnp.float32),
    )(xT, m, s1, s2, c1r, c1c, c2r, c2c)
    return out.reshape(_S)


# speedup vs baseline: 1.1031x; 1.1031x over previous
import jax
import jax.numpy as jnp
from jax.experimental import pallas as pl
from jax.experimental.pallas import tpu as pltpu

_S = 8192
_B = 1024
_f32 = jnp.float32


def _sum32(a):
    acc = (a[0:8] + a[8:16]) + a[16:24]
    acc = acc + a[24:32]
    u = acc[0:4] + acc[4:8]
    v = u[0:2] + u[2:4]
    return v[0:1] + v[1:2]


def _km_block(p1_ref, p2_ref, x_ref, m_ref, s1_ref, s2_ref, c1s_ref, c2s_ref,
              c1c_ref, c2c_ref, out_ref, uu_s, ll_s, luu_s, lll_s, ruu_s,
              rll_s):
    B = out_ref.shape[1]
    a1 = jnp.abs(s1_ref[...])
    a2 = jnp.abs(s2_ref[...])
    slo = jnp.minimum(a1, a2) + _f32(1e-6)
    shi = jnp.maximum(a1, a2) + _f32(1e-6)
    m = m_ref[...]

    es = []
    eb = []
    for k in range(8):
        xkb = jnp.broadcast_to(x_ref[k:k + 1, :], (32, B))
        d = xkb - jnp.broadcast_to(m[:, k:k + 1], (32, B))
        qs = d / jnp.broadcast_to(slo[:, k:k + 1], (32, B))
        qb = d / jnp.broadcast_to(shi[:, k:k + 1], (32, B))
        es.append(jnp.exp((qs * qs) * _f32(-0.5)))
        eb.append(jnp.exp((qb * qb) * _f32(-0.5)))

    def prod8(e):
        u0 = e[0] * e[4]
        u1 = e[1] * e[5]
        u2 = e[2] * e[6]
        u3 = e[3] * e[7]
        return (u0 * u2) * (u1 * u3)

    ll_s[...] = prod8(es)
    uu_s[...] = prod8(eb)

    for t in range(32):
        r1 = p1_ref[t]
        r2 = p2_ref[t]
        luu_s[t:t + 1, :] = uu_s[pl.ds(r1, 1), :]
        lll_s[t:t + 1, :] = ll_s[pl.ds(r1, 1), :]
        ruu_s[t:t + 1, :] = uu_s[pl.ds(r2, 1), :]
        rll_s[t:t + 1, :] = ll_s[pl.ds(r2, 1), :]

    L_UU = luu_s[...]
    L_LL = lll_s[...]
    R_UU = ruu_s[...]
    R_LL = rll_s[...]

    c1s_b = jnp.broadcast_to(c1s_ref[...], (32, B))
    c2s_b = jnp.broadcast_to(c2s_ref[...], (32, B))
    s0 = _sum32(c1s_b * L_LL)
    t0 = _sum32(L_LL)
    s0r = _sum32(c2s_b * R_UU)
    t0r = _sum32(R_UU)
    dL = L_UU - L_LL
    dR = R_LL - R_UU
    ndL = c1s_b * dL
    ndR = c2s_b * dR
    q = s0 / t0
    qr = s0r / t0r

    cn = jnp.zeros((1, B), _f32)
    cd = jnp.zeros((1, B), _f32)
    cmn = jnp.full((1, B), jnp.inf, _f32)
    bestv = jnp.full((1, B), jnp.inf, _f32)
    besti = jnp.zeros((1, B), jnp.int32)
    for t in range(32):
        cn = cn + ndL[t:t + 1]
        cd = cd + dL[t:t + 1]
        ratio = (s0 + cn) / (t0 + cd)
        cmn = jnp.minimum(cmn, ratio)
        lout = jnp.minimum(cmn, q)
        keepv = (bestv < lout) | (bestv != bestv)
        keepi = keepv | (bestv == lout)
        bestv = jnp.where(keepv, bestv, lout)
        besti = jnp.where(keepi, besti, jnp.full((1, B), t, jnp.int32))
    L_loc = besti

    cn = jnp.zeros((1, B), _f32)
    cd = jnp.zeros((1, B), _f32)
    cmx = jnp.full((1, B), -jnp.inf, _f32)
    bestv = jnp.full((1, B), -jnp.inf, _f32)
    besti = jnp.zeros((1, B), jnp.int32)
    for t in range(32):
        cn = cn + ndR[t:t + 1]
        cd = cd + dR[t:t + 1]
        ratio = (s0r + cn) / (t0r + cd)
        cmx = jnp.maximum(cmx, ratio)
        rout = jnp.maximum(cmx, qr)
        keepv = (bestv > rout) | (bestv != bestv)
        keepi = keepv | (bestv == rout)
        bestv = jnp.where(keepv, bestv, rout)
        besti = jnp.where(keepi, besti, jnp.full((1, B), t, jnp.int32))
    R_loc = besti

    rowi = jax.lax.broadcasted_iota(jnp.int32, (32, B), 0)
    selL = jnp.where(rowi <= jnp.broadcast_to(L_loc, (32, B)), L_UU, L_LL)
    selR = jnp.where(rowi <= jnp.broadcast_to(R_loc, (32, B)), R_LL, R_UU)
    c1n_b = jnp.broadcast_to(c1c_ref[...], (32, B))
    c2n_b = jnp.broadcast_to(c2c_ref[...], (32, B))
    out_left = _sum32(c1n_b * selL) / _sum32(selL)
    out_right = _sum32(c2n_b * selR) / _sum32(selR)
    out_ref[...] = (out_right + out_left) / _f32(2.0)


def kernel(input_data, FRB_weights, c1, c2):
    xT = input_data.T
    m = FRB_weights[0:256].reshape(32, 8)
    s1 = FRB_weights[1:257].reshape(32, 8)
    s2 = FRB_weights[2:258].reshape(32, 8)
    p1 = jnp.argsort(c1).astype(jnp.int32)
    p2 = jnp.argsort(c2).astype(jnp.int32)
    c1s = c1[p1].reshape(32, 1)
    c2s = c2[p2].reshape(32, 1)
    c1c = c1.reshape(32, 1)
    c2c = c2.reshape(32, 1)
    rep = pl.BlockSpec((32, 8), lambda i, p1, p2: (0, 0))
    col = pl.BlockSpec((32, 1), lambda i, p1, p2: (0, 0))
    grid_spec = pltpu.PrefetchScalarGridSpec(
        num_scalar_prefetch=2,
        grid=(_S // _B,),
        in_specs=[
            pl.BlockSpec((8, _B), lambda i, p1, p2: (0, i)),
            rep, rep, rep,
            col, col, col, col,
        ],
        out_specs=pl.BlockSpec((1, _B), lambda i, p1, p2: (0, i)),
        scratch_shapes=[pltpu.VMEM((32, _B), jnp.float32) for _ in range(6)],
    )
    out = pl.pallas_call(
        _km_block,
        grid_spec=grid_spec,
        out_shape=jax.ShapeDtypeStruct((1, _S), jnp.float32),
    )(p1, p2, xT, m, s1, s2, c1s, c2s, c1c, c2c)
    return out.reshape(_S)


# speedup vs baseline: 1.3233x; 1.1996x over previous
import jax
import jax.numpy as jnp
from jax.experimental import pallas as pl
from jax.experimental.pallas import tpu as pltpu

_S = 8192
_B = 1024
_G = 8
_B8 = _B // _G
_f32 = jnp.float32


def _sum32(a):
    acc = (a[0:64] + a[64:128]) + a[128:192]
    acc = acc + a[192:256]
    u = acc[0:32] + acc[32:64]
    v = u[0:16] + u[16:32]
    return v[0:8] + v[8:16]


def _km_block(p1_ref, p2_ref, x_ref, m_ref, s1_ref, s2_ref, c1s_ref, c2s_ref,
              c1c_ref, c2c_ref, out_ref, uu_s, ll_s, luu_s, lll_s, ruu_s,
              rll_s):
    a1 = jnp.abs(s1_ref[...])
    a2 = jnp.abs(s2_ref[...])
    slo = jnp.minimum(a1, a2) + _f32(1e-6)
    shi = jnp.maximum(a1, a2) + _f32(1e-6)
    m = m_ref[...]

    es = []
    eb = []
    for k in range(8):
        xk = x_ref[k * _G:(k + 1) * _G, :]
        xkb = jnp.broadcast_to(xk[None, :, :], (32, _G, _B8)).reshape(256, _B8)
        d = xkb - jnp.broadcast_to(m[:, k:k + 1], (256, _B8))
        qs = d / jnp.broadcast_to(slo[:, k:k + 1], (256, _B8))
        qb = d / jnp.broadcast_to(shi[:, k:k + 1], (256, _B8))
        es.append(jnp.exp((qs * qs) * _f32(-0.5)))
        eb.append(jnp.exp((qb * qb) * _f32(-0.5)))

    def prod8(e):
        u0 = e[0] * e[4]
        u1 = e[1] * e[5]
        u2 = e[2] * e[6]
        u3 = e[3] * e[7]
        return (u0 * u2) * (u1 * u3)

    ll_s[...] = prod8(es)
    uu_s[...] = prod8(eb)

    for t in range(32):
        r1 = p1_ref[t] * _G
        r2 = p2_ref[t] * _G
        luu_s[t * _G:(t + 1) * _G, :] = uu_s[pl.ds(r1, _G), :]
        lll_s[t * _G:(t + 1) * _G, :] = ll_s[pl.ds(r1, _G), :]
        ruu_s[t * _G:(t + 1) * _G, :] = uu_s[pl.ds(r2, _G), :]
        rll_s[t * _G:(t + 1) * _G, :] = ll_s[pl.ds(r2, _G), :]

    L_UU = luu_s[...]
    L_LL = lll_s[...]
    R_UU = ruu_s[...]
    R_LL = rll_s[...]

    c1s_b = jnp.broadcast_to(c1s_ref[...], (256, _B8))
    c2s_b = jnp.broadcast_to(c2s_ref[...], (256, _B8))
    s0 = _sum32(c1s_b * L_LL)
    t0 = _sum32(L_LL)
    s0r = _sum32(c2s_b * R_UU)
    t0r = _sum32(R_UU)
    dL = L_UU - L_LL
    dR = R_LL - R_UU
    ndL = c1s_b * dL
    ndR = c2s_b * dR
    q = s0 / t0
    qr = s0r / t0r

    cn = jnp.zeros((_G, _B8), _f32)
    cd = jnp.zeros((_G, _B8), _f32)
    cmn = jnp.full((_G, _B8), jnp.inf, _f32)
    bestv = jnp.full((_G, _B8), jnp.inf, _f32)
    besti = jnp.zeros((_G, _B8), jnp.int32)
    for t in range(32):
        cn = cn + ndL[t * _G:(t + 1) * _G]
        cd = cd + dL[t * _G:(t + 1) * _G]
        ratio = (s0 + cn) / (t0 + cd)
        cmn = jnp.minimum(cmn, ratio)
        lout = jnp.minimum(cmn, q)
        keepv = (bestv < lout) | (bestv != bestv)
        keepi = keepv | (bestv == lout)
        bestv = jnp.where(keepv, bestv, lout)
        besti = jnp.where(keepi, besti, jnp.full((_G, _B8), t, jnp.int32))
    L_loc = besti

    cn = jnp.zeros((_G, _B8), _f32)
    cd = jnp.zeros((_G, _B8), _f32)
    cmx = jnp.full((_G, _B8), -jnp.inf, _f32)
    bestv = jnp.full((_G, _B8), -jnp.inf, _f32)
    besti = jnp.zeros((_G, _B8), jnp.int32)
    for t in range(32):
        cn = cn + ndR[t * _G:(t + 1) * _G]
        cd = cd + dR[t * _G:(t + 1) * _G]
        ratio = (s0r + cn) / (t0r + cd)
        cmx = jnp.maximum(cmx, ratio)
        rout = jnp.maximum(cmx, qr)
        keepv = (bestv > rout) | (bestv != bestv)
        keepi = keepv | (bestv == rout)
        bestv = jnp.where(keepv, bestv, rout)
        besti = jnp.where(keepi, besti, jnp.full((_G, _B8), t, jnp.int32))
    R_loc = besti

    rulei = jax.lax.broadcasted_iota(jnp.int32, (32, _G, _B8), 0).reshape(256, _B8)
    L_loc_b = jnp.broadcast_to(L_loc[None, :, :], (32, _G, _B8)).reshape(256, _B8)
    R_loc_b = jnp.broadcast_to(R_loc[None, :, :], (32, _G, _B8)).reshape(256, _B8)
    selL = jnp.where(rulei <= L_loc_b, L_UU, L_LL)
    selR = jnp.where(rulei <= R_loc_b, R_LL, R_UU)
    c1n_b = jnp.broadcast_to(c1c_ref[...], (256, _B8))
    c2n_b = jnp.broadcast_to(c2c_ref[...], (256, _B8))
    out_left = _sum32(c1n_b * selL) / _sum32(selL)
    out_right = _sum32(c2n_b * selR) / _sum32(selR)
    out_ref[...] = (out_right + out_left) / _f32(2.0)


def kernel(input_data, FRB_weights, c1, c2):
    nblk = _S // _B
    xp = (input_data.T.reshape(8, nblk, _G, _B8)
          .transpose(1, 0, 2, 3).reshape(nblk * 8 * _G, _B8))
    m = jnp.repeat(FRB_weights[0:256].reshape(32, 8), _G, axis=0)
    s1 = jnp.repeat(FRB_weights[1:257].reshape(32, 8), _G, axis=0)
    s2 = jnp.repeat(FRB_weights[2:258].reshape(32, 8), _G, axis=0)
    p1 = jnp.argsort(c1).astype(jnp.int32)
    p2 = jnp.argsort(c2).astype(jnp.int32)
    c1s = jnp.repeat(c1[p1], _G).reshape(256, 1)
    c2s = jnp.repeat(c2[p2], _G).reshape(256, 1)
    c1c = jnp.repeat(c1, _G).reshape(256, 1)
    c2c = jnp.repeat(c2, _G).reshape(256, 1)
    rep = pl.BlockSpec((256, 8), lambda i, p1, p2: (0, 0))
    col = pl.BlockSpec((256, 1), lambda i, p1, p2: (0, 0))
    grid_spec = pltpu.PrefetchScalarGridSpec(
        num_scalar_prefetch=2,
        grid=(nblk,),
        in_specs=[
            pl.BlockSpec((8 * _G, _B8), lambda i, p1, p2: (i, 0)),
            rep, rep, rep,
            col, col, col, col,
        ],
        out_specs=pl.BlockSpec((_G, _B8), lambda i, p1, p2: (i, 0)),
        scratch_shapes=[pltpu.VMEM((256, _B8), jnp.float32) for _ in range(6)],
    )
    out = pl.pallas_call(
        _km_block,
        grid_spec=grid_spec,
        out_shape=jax.ShapeDtypeStruct((nblk * _G, _B8), jnp.float32),
    )(p1, p2, xp, m, s1, s2, c1s, c2s, c1c, c2c)
    return out.reshape(_S)


# speedup vs baseline: 1.4021x; 1.0595x over previous
import jax
import jax.numpy as jnp
from jax.experimental import pallas as pl
from jax.experimental.pallas import tpu as pltpu

_S = 8192
_B = 2048
_G = 8
_B8 = _B // _G
_f32 = jnp.float32


def _sum32(a):
    acc = (a[0:64] + a[64:128]) + a[128:192]
    acc = acc + a[192:256]
    u = acc[0:32] + acc[32:64]
    v = u[0:16] + u[16:32]
    return v[0:8] + v[8:16]


def _km_block(p1_ref, p2_ref, x_ref, m_ref, s1_ref, s2_ref, c1s_ref, c2s_ref,
              c1c_ref, c2c_ref, out_ref, uu_s, ll_s, luu_s, lll_s, ruu_s,
              rll_s):
    a1 = jnp.abs(s1_ref[...])
    a2 = jnp.abs(s2_ref[...])
    slo = jnp.minimum(a1, a2) + _f32(1e-6)
    shi = jnp.maximum(a1, a2) + _f32(1e-6)
    m = m_ref[...]

    es = []
    eb = []
    for k in range(8):
        xk = x_ref[k * _G:(k + 1) * _G, :]
        xkb = jnp.broadcast_to(xk[None, :, :], (32, _G, _B8)).reshape(256, _B8)
        d = xkb - jnp.broadcast_to(m[:, k:k + 1], (256, _B8))
        qs = d / jnp.broadcast_to(slo[:, k:k + 1], (256, _B8))
        qb = d / jnp.broadcast_to(shi[:, k:k + 1], (256, _B8))
        es.append(jnp.exp((qs * qs) * _f32(-0.5)))
        eb.append(jnp.exp((qb * qb) * _f32(-0.5)))

    def prod8(e):
        u0 = e[0] * e[4]
        u1 = e[1] * e[5]
        u2 = e[2] * e[6]
        u3 = e[3] * e[7]
        return (u0 * u2) * (u1 * u3)

    ll_s[...] = prod8(es)
    uu_s[...] = prod8(eb)

    for t in range(32):
        r1 = p1_ref[t] * _G
        r2 = p2_ref[t] * _G
        luu_s[t * _G:(t + 1) * _G, :] = uu_s[pl.ds(r1, _G), :]
        lll_s[t * _G:(t + 1) * _G, :] = ll_s[pl.ds(r1, _G), :]
        ruu_s[t * _G:(t + 1) * _G, :] = uu_s[pl.ds(r2, _G), :]
        rll_s[t * _G:(t + 1) * _G, :] = ll_s[pl.ds(r2, _G), :]

    L_UU = luu_s[...]
    L_LL = lll_s[...]
    R_UU = ruu_s[...]
    R_LL = rll_s[...]

    c1s_b = jnp.broadcast_to(c1s_ref[...], (256, _B8))
    c2s_b = jnp.broadcast_to(c2s_ref[...], (256, _B8))
    s0 = _sum32(c1s_b * L_LL)
    t0 = _sum32(L_LL)
    s0r = _sum32(c2s_b * R_UU)
    t0r = _sum32(R_UU)
    dL = L_UU - L_LL
    dR = R_LL - R_UU
    ndL = c1s_b * dL
    ndR = c2s_b * dR
    q = s0 / t0
    qr = s0r / t0r

    cn = jnp.zeros((_G, _B8), _f32)
    cd = jnp.zeros((_G, _B8), _f32)
    cmn = jnp.full((_G, _B8), jnp.inf, _f32)
    bestv = jnp.full((_G, _B8), jnp.inf, _f32)
    besti = jnp.zeros((_G, _B8), jnp.int32)
    for t in range(32):
        cn = cn + ndL[t * _G:(t + 1) * _G]
        cd = cd + dL[t * _G:(t + 1) * _G]
        ratio = (s0 + cn) / (t0 + cd)
        cmn = jnp.minimum(cmn, ratio)
        lout = jnp.minimum(cmn, q)
        keepv = (bestv < lout) | (bestv != bestv)
        keepi = keepv | (bestv == lout)
        bestv = jnp.where(keepv, bestv, lout)
        besti = jnp.where(keepi, besti, jnp.full((_G, _B8), t, jnp.int32))
    L_loc = besti

    cn = jnp.zeros((_G, _B8), _f32)
    cd = jnp.zeros((_G, _B8), _f32)
    cmx = jnp.full((_G, _B8), -jnp.inf, _f32)
    bestv = jnp.full((_G, _B8), -jnp.inf, _f32)
    besti = jnp.zeros((_G, _B8), jnp.int32)
    for t in range(32):
        cn = cn + ndR[t * _G:(t + 1) * _G]
        cd = cd + dR[t * _G:(t + 1) * _G]
        ratio = (s0r + cn) / (t0r + cd)
        cmx = jnp.maximum(cmx, ratio)
        rout = jnp.maximum(cmx, qr)
        keepv = (bestv > rout) | (bestv != bestv)
        keepi = keepv | (bestv == rout)
        bestv = jnp.where(keepv, bestv, rout)
        besti = jnp.where(keepi, besti, jnp.full((_G, _B8), t, jnp.int32))
    R_loc = besti

    rulei = jax.lax.broadcasted_iota(jnp.int32, (32, _G, _B8), 0).reshape(256, _B8)
    L_loc_b = jnp.broadcast_to(L_loc[None, :, :], (32, _G, _B8)).reshape(256, _B8)
    R_loc_b = jnp.broadcast_to(R_loc[None, :, :], (32, _G, _B8)).reshape(256, _B8)
    selL = jnp.where(rulei <= L_loc_b, L_UU, L_LL)
    selR = jnp.where(rulei <= R_loc_b, R_LL, R_UU)
    c1n_b = jnp.broadcast_to(c1c_ref[...], (256, _B8))
    c2n_b = jnp.broadcast_to(c2c_ref[...], (256, _B8))
    out_left = _sum32(c1n_b * selL) / _sum32(selL)
    out_right = _sum32(c2n_b * selR) / _sum32(selR)
    out_ref[...] = (out_right + out_left) / _f32(2.0)


def kernel(input_data, FRB_weights, c1, c2):
    nblk = _S // _B
    xp = (input_data.T.reshape(8, nblk, _G, _B8)
          .transpose(1, 0, 2, 3).reshape(nblk * 8 * _G, _B8))
    m = jnp.repeat(FRB_weights[0:256].reshape(32, 8), _G, axis=0)
    s1 = jnp.repeat(FRB_weights[1:257].reshape(32, 8), _G, axis=0)
    s2 = jnp.repeat(FRB_weights[2:258].reshape(32, 8), _G, axis=0)
    p1 = jnp.argsort(c1).astype(jnp.int32)
    p2 = jnp.argsort(c2).astype(jnp.int32)
    c1s = jnp.repeat(c1[p1], _G).reshape(256, 1)
    c2s = jnp.repeat(c2[p2], _G).reshape(256, 1)
    c1c = jnp.repeat(c1, _G).reshape(256, 1)
    c2c = jnp.repeat(c2, _G).reshape(256, 1)
    rep = pl.BlockSpec((256, 8), lambda i, p1, p2: (0, 0))
    col = pl.BlockSpec((256, 1), lambda i, p1, p2: (0, 0))
    grid_spec = pltpu.PrefetchScalarGridSpec(
        num_scalar_prefetch=2,
        grid=(nblk,),
        in_specs=[
            pl.BlockSpec((8 * _G, _B8), lambda i, p1, p2: (i, 0)),
            rep, rep, rep,
            col, col, col, col,
        ],
        out_specs=pl.BlockSpec((_G, _B8), lambda i, p1, p2: (i, 0)),
        scratch_shapes=[pltpu.VMEM((256, _B8), jnp.float32) for _ in range(6)],
    )
    out = pl.pallas_call(
        _km_block,
        grid_spec=grid_spec,
        out_shape=jax.ShapeDtypeStruct((nblk * _G, _B8), jnp.float32),
    )(p1, p2, xp, m, s1, s2, c1s, c2s, c1c, c2c)
    return out.reshape(_S)


# speedup vs baseline: 1.4340x; 1.0227x over previous
import jax
import jax.numpy as jnp
from jax.experimental import pallas as pl
from jax.experimental.pallas import tpu as pltpu

_S = 8192
_B = 4096
_G = 8
_B8 = _B // _G
_f32 = jnp.float32


def _sum32(a):
    acc = (a[0:64] + a[64:128]) + a[128:192]
    acc = acc + a[192:256]
    u = acc[0:32] + acc[32:64]
    v = u[0:16] + u[16:32]
    return v[0:8] + v[8:16]


def _km_block(p1_ref, p2_ref, x_ref, m_ref, s1_ref, s2_ref, c1s_ref, c2s_ref,
              c1c_ref, c2c_ref, out_ref, uu_s, ll_s, luu_s, lll_s, ruu_s,
              rll_s):
    a1 = jnp.abs(s1_ref[...])
    a2 = jnp.abs(s2_ref[...])
    slo = jnp.minimum(a1, a2) + _f32(1e-6)
    shi = jnp.maximum(a1, a2) + _f32(1e-6)
    m = m_ref[...]

    es = []
    eb = []
    for k in range(8):
        xk = x_ref[k * _G:(k + 1) * _G, :]
        xkb = jnp.broadcast_to(xk[None, :, :], (32, _G, _B8)).reshape(256, _B8)
        d = xkb - jnp.broadcast_to(m[:, k:k + 1], (256, _B8))
        qs = d / jnp.broadcast_to(slo[:, k:k + 1], (256, _B8))
        qb = d / jnp.broadcast_to(shi[:, k:k + 1], (256, _B8))
        es.append(jnp.exp((qs * qs) * _f32(-0.5)))
        eb.append(jnp.exp((qb * qb) * _f32(-0.5)))

    def prod8(e):
        u0 = e[0] * e[4]
        u1 = e[1] * e[5]
        u2 = e[2] * e[6]
        u3 = e[3] * e[7]
        return (u0 * u2) * (u1 * u3)

    ll_s[...] = prod8(es)
    uu_s[...] = prod8(eb)

    for t in range(32):
        r1 = p1_ref[t] * _G
        r2 = p2_ref[t] * _G
        luu_s[t * _G:(t + 1) * _G, :] = uu_s[pl.ds(r1, _G), :]
        lll_s[t * _G:(t + 1) * _G, :] = ll_s[pl.ds(r1, _G), :]
        ruu_s[t * _G:(t + 1) * _G, :] = uu_s[pl.ds(r2, _G), :]
        rll_s[t * _G:(t + 1) * _G, :] = ll_s[pl.ds(r2, _G), :]

    L_UU = luu_s[...]
    L_LL = lll_s[...]
    R_UU = ruu_s[...]
    R_LL = rll_s[...]

    c1s_b = jnp.broadcast_to(c1s_ref[...], (256, _B8))
    c2s_b = jnp.broadcast_to(c2s_ref[...], (256, _B8))
    s0 = _sum32(c1s_b * L_LL)
    t0 = _sum32(L_LL)
    s0r = _sum32(c2s_b * R_UU)
    t0r = _sum32(R_UU)
    dL = L_UU - L_LL
    dR = R_LL - R_UU
    ndL = c1s_b * dL
    ndR = c2s_b * dR
    q = s0 / t0
    qr = s0r / t0r

    cn = jnp.zeros((_G, _B8), _f32)
    cd = jnp.zeros((_G, _B8), _f32)
    cmn = jnp.full((_G, _B8), jnp.inf, _f32)
    bestv = jnp.full((_G, _B8), jnp.inf, _f32)
    besti = jnp.zeros((_G, _B8), jnp.int32)
    for t in range(32):
        cn = cn + ndL[t * _G:(t + 1) * _G]
        cd = cd + dL[t * _G:(t + 1) * _G]
        ratio = (s0 + cn) / (t0 + cd)
        cmn = jnp.minimum(cmn, ratio)
        lout = jnp.minimum(cmn, q)
        keepv = (bestv < lout) | (bestv != bestv)
        keepi = keepv | (bestv == lout)
        bestv = jnp.where(keepv, bestv, lout)
        besti = jnp.where(keepi, besti, jnp.full((_G, _B8), t, jnp.int32))
    L_loc = besti

    cn = jnp.zeros((_G, _B8), _f32)
    cd = jnp.zeros((_G, _B8), _f32)
    cmx = jnp.full((_G, _B8), -jnp.inf, _f32)
    bestv = jnp.full((_G, _B8), -jnp.inf, _f32)
    besti = jnp.zeros((_G, _B8), jnp.int32)
    for t in range(32):
        cn = cn + ndR[t * _G:(t + 1) * _G]
        cd = cd + dR[t * _G:(t + 1) * _G]
        ratio = (s0r + cn) / (t0r + cd)
        cmx = jnp.maximum(cmx, ratio)
        rout = jnp.maximum(cmx, qr)
        keepv = (bestv > rout) | (bestv != bestv)
        keepi = keepv | (bestv == rout)
        bestv = jnp.where(keepv, bestv, rout)
        besti = jnp.where(keepi, besti, jnp.full((_G, _B8), t, jnp.int32))
    R_loc = besti

    rulei = jax.lax.broadcasted_iota(jnp.int32, (32, _G, _B8), 0).reshape(256, _B8)
    L_loc_b = jnp.broadcast_to(L_loc[None, :, :], (32, _G, _B8)).reshape(256, _B8)
    R_loc_b = jnp.broadcast_to(R_loc[None, :, :], (32, _G, _B8)).reshape(256, _B8)
    selL = jnp.where(rulei <= L_loc_b, L_UU, L_LL)
    selR = jnp.where(rulei <= R_loc_b, R_LL, R_UU)
    c1n_b = jnp.broadcast_to(c1c_ref[...], (256, _B8))
    c2n_b = jnp.broadcast_to(c2c_ref[...], (256, _B8))
    out_left = _sum32(c1n_b * selL) / _sum32(selL)
    out_right = _sum32(c2n_b * selR) / _sum32(selR)
    out_ref[...] = (out_right + out_left) / _f32(2.0)


def kernel(input_data, FRB_weights, c1, c2):
    nblk = _S // _B
    xp = (input_data.T.reshape(8, nblk, _G, _B8)
          .transpose(1, 0, 2, 3).reshape(nblk * 8 * _G, _B8))
    m = jnp.repeat(FRB_weights[0:256].reshape(32, 8), _G, axis=0)
    s1 = jnp.repeat(FRB_weights[1:257].reshape(32, 8), _G, axis=0)
    s2 = jnp.repeat(FRB_weights[2:258].reshape(32, 8), _G, axis=0)
    p1 = jnp.argsort(c1).astype(jnp.int32)
    p2 = jnp.argsort(c2).astype(jnp.int32)
    c1s = jnp.repeat(c1[p1], _G).reshape(256, 1)
    c2s = jnp.repeat(c2[p2], _G).reshape(256, 1)
    c1c = jnp.repeat(c1, _G).reshape(256, 1)
    c2c = jnp.repeat(c2, _G).reshape(256, 1)
    rep = pl.BlockSpec((256, 8), lambda i, p1, p2: (0, 0))
    col = pl.BlockSpec((256, 1), lambda i, p1, p2: (0, 0))
    grid_spec = pltpu.PrefetchScalarGridSpec(
        num_scalar_prefetch=2,
        grid=(nblk,),
        in_specs=[
            pl.BlockSpec((8 * _G, _B8), lambda i, p1, p2: (i, 0)),
            rep, rep, rep,
            col, col, col, col,
        ],
        out_specs=pl.BlockSpec((_G, _B8), lambda i, p1, p2: (i, 0)),
        scratch_shapes=[pltpu.VMEM((256, _B8), jnp.float32) for _ in range(6)],
    )
    out = pl.pallas_call(
        _km_block,
        grid_spec=grid_spec,
        out_shape=jax.ShapeDtypeStruct((nblk * _G, _B8), jnp.float32),
    )(p1, p2, xp, m, s1, s2, c1s, c2s, c1c, c2c)
    return out.reshape(_S)
